# Initial kernel scaffold; baseline (speedup 1.0000x reference)
import jax
import jax.numpy as jnp
from jax.experimental import pallas as pl


def kernel(xyz, x, pos_in, batch, pretrain_global, params):
    raise NotImplementedError("write your pallas kernel here")


# speedup vs baseline: 4.5762x; 4.5762x over previous
import functools

import jax
import jax.numpy as jnp
from jax import lax
from jax.experimental import pallas as pl
from jax.experimental.pallas import tpu as pltpu
from jax.experimental.pallas import tpu_sc as plsc

N = 10000
B = 4
K = 20
NPAD = 10240
BQ = 80
BN_ = 80
EBLK = BN_ * K
E = N * K
EPAD = 204800
BIG1 = 1e30
BIG2 = 1e31
NBIG = 1 << 30

SC_NC = 2
SC_NS = 16
SC_WORKERS = SC_NC * SC_NS
SC_PER_W = EPAD // SC_WORKERS
SC_CH = 128
SC_ITERS = SC_PER_W // SC_CH


def _knn_body(qpos_ref, post_ref, batchrow_ref, qbatch_ref, out_ref):
    i = pl.program_id(0)
    q = qpos_ref[...]
    pt = post_ref[...]
    d = ((q[:, 0:1] - pt[0:1, :]) ** 2 + (q[:, 1:2] - pt[1:2, :]) ** 2
         + (q[:, 2:3] - pt[2:3, :]) ** 2)
    lanes = lax.broadcasted_iota(jnp.int32, (BQ, NPAD), 1)
    qidx = i * BQ + lax.broadcasted_iota(jnp.int32, (BQ, 1), 0)
    valid = (batchrow_ref[...] == qbatch_ref[...]) & (lanes != qidx)
    dm = jnp.where(valid, d, BIG1)
    cols = []
    for _ in range(K):
        m = jnp.min(dm, axis=1, keepdims=True)
        cand = jnp.where(dm == m, lanes, NBIG)
        idx = jnp.min(cand, axis=1, keepdims=True)
        cols.append(idx)
        dm = jnp.where(lanes == idx, BIG2, dm)
    out_ref[...] = jnp.concatenate(cols, axis=1)


def _knn(pos_in, batch_f):
    pos8 = jnp.pad(pos_in, ((0, 0), (0, 5)))
    post = jnp.pad(pos_in, ((0, NPAD - N), (0, 5))).T
    batchrow = jnp.pad(batch_f[None, :], ((0, 0), (0, NPAD - N)),
                       constant_values=-1.0)
    qbatch = batch_f[:, None]
    return pl.pallas_call(
        _knn_body,
        grid=(N // BQ,),
        in_specs=[
            pl.BlockSpec((BQ, 8), lambda i: (i, 0)),
            pl.BlockSpec((8, NPAD), lambda i: (0, 0)),
            pl.BlockSpec((1, NPAD), lambda i: (0, 0)),
            pl.BlockSpec((BQ, 1), lambda i: (i, 0)),
        ],
        out_specs=pl.BlockSpec((BQ, K), lambda i: (i, 0)),
        out_shape=jax.ShapeDtypeStruct((N, K), jnp.int32),
    )(pos8, post, batchrow, qbatch)


def _pq1_body(xin_ref, w0_ref, b0_ref, w1_ref, p_ref, q_ref):
    x0 = jnp.dot(xin_ref[...], w0_ref[...],
                 preferred_element_type=jnp.float32) + b0_ref[...]
    d = w1_ref.shape[0] // 2
    wa = w1_ref[:d, :]
    wb = w1_ref[d:, :]
    p_ref[...] = jnp.dot(x0, wa - wb, preferred_element_type=jnp.float32)
    q_ref[...] = jnp.dot(x0, wb, preferred_element_type=jnp.float32)


def _pq1(xin, w0p, b0, w1, h):
    br = 400
    return pl.pallas_call(
        _pq1_body,
        grid=(N // br,),
        in_specs=[
            pl.BlockSpec((br, 16), lambda i: (i, 0)),
            pl.BlockSpec(w0p.shape, lambda i: (0, 0)),
            pl.BlockSpec((1, w0p.shape[1]), lambda i: (0, 0)),
            pl.BlockSpec(w1.shape, lambda i: (0, 0)),
        ],
        out_specs=[
            pl.BlockSpec((br, h), lambda i: (i, 0)),
            pl.BlockSpec((br, h), lambda i: (i, 0)),
        ],
        out_shape=[
            jax.ShapeDtypeStruct((N, h), jnp.float32),
            jax.ShapeDtypeStruct((N, h), jnp.float32),
        ],
    )(xin, w0p, b0, w1)


def _pq23_body(s_ref, st_ref, g_ref, be_ref, w1_ref, p_ref, q_ref):
    mu = st_ref[0:1, :] / N
    var = st_ref[1:2, :] / N - mu * mu
    inv = lax.rsqrt(var + 1e-5)
    xn = (s_ref[...] - mu) * inv * g_ref[...] + be_ref[...]
    d = w1_ref.shape[0] // 2
    wa = w1_ref[:d, :]
    wb = w1_ref[d:, :]
    p_ref[...] = jnp.dot(xn, wa - wb, preferred_element_type=jnp.float32)
    q_ref[...] = jnp.dot(xn, wb, preferred_element_type=jnp.float32)


def _pq23(s, stats, g, be, w1, h):
    br = 400
    din = s.shape[1]
    return pl.pallas_call(
        _pq23_body,
        grid=(N // br,),
        in_specs=[
            pl.BlockSpec((br, din), lambda i: (i, 0)),
            pl.BlockSpec((8, din), lambda i: (0, 0)),
            pl.BlockSpec((1, din), lambda i: (0, 0)),
            pl.BlockSpec((1, din), lambda i: (0, 0)),
            pl.BlockSpec(w1.shape, lambda i: (0, 0)),
        ],
        out_specs=[
            pl.BlockSpec((br, h), lambda i: (i, 0)),
            pl.BlockSpec((br, h), lambda i: (i, 0)),
        ],
        out_shape=[
            jax.ShapeDtypeStruct((N, h), jnp.float32),
            jax.ShapeDtypeStruct((N, h), jnp.float32),
        ],
    )(s, stats, g, be, w1)


def _sc_gather(table, idx_pad, h):
    mesh = plsc.VectorSubcoreMesh(core_axis_name="c", subcore_axis_name="s",
                                  num_cores=SC_NC, num_subcores=SC_NS)

    @functools.partial(
        pl.kernel,
        out_type=jax.ShapeDtypeStruct((EPAD, h), jnp.float32),
        mesh=mesh,
        scratch_types=[
            pltpu.VMEM((SC_CH,), jnp.int32),
            pltpu.VMEM((SC_CH, h), jnp.float32),
            pltpu.SemaphoreType.DMA,
        ],
    )
    def k(table_hbm, idx_hbm, out_hbm, idx_v, rows_v, sem):
        wid = lax.axis_index("s") * SC_NC + lax.axis_index("c")
        base = wid * SC_PER_W

        def body(t, carry):
            off = base + t * SC_CH
            pltpu.sync_copy(idx_hbm.at[pl.ds(off, SC_CH)], idx_v)
            pltpu.async_copy(table_hbm.at[idx_v], rows_v, sem).wait()
            pltpu.sync_copy(rows_v, out_hbm.at[pl.ds(off, SC_CH)])
            return carry

        lax.fori_loop(0, SC_ITERS, body, 0)

    return k(table, idx_pad)


def _edge_body(p_ref, qg_ref, b1_ref, w2_ref, b2_ref, s_ref, st_ref):
    i = pl.program_id(0)
    hpad = p_ref.shape[1]
    h = w2_ref.shape[1]
    p = p_ref[...]
    qg = qg_ref[...].reshape(BN_, K, hpad)
    a = jax.nn.relu(qg + p[:, None, :] + b1_ref[...][None])
    m = jnp.dot(a.reshape(EBLK, hpad), w2_ref[...],
                preferred_element_type=jnp.float32)
    s = jax.nn.relu(jnp.max(m.reshape(BN_, K, h), axis=1) + b2_ref[...])
    s_ref[...] = s

    @pl.when(i == 0)
    def _():
        st_ref[...] = jnp.zeros_like(st_ref)

    st_ref[0:1, :] += jnp.sum(s, axis=0, keepdims=True)
    st_ref[1:2, :] += jnp.sum(s * s, axis=0, keepdims=True)


def _edge(p, qg, b1, w2, b2, h, hpad):
    return pl.pallas_call(
        _edge_body,
        grid=(N // BN_,),
        in_specs=[
            pl.BlockSpec((BN_, hpad), lambda i: (i, 0)),
            pl.BlockSpec((EBLK, hpad), lambda i: (i, 0)),
            pl.BlockSpec((1, hpad), lambda i: (0, 0)),
            pl.BlockSpec((hpad, h), lambda i: (0, 0)),
            pl.BlockSpec((1, h), lambda i: (0, 0)),
        ],
        out_specs=[
            pl.BlockSpec((BN_, h), lambda i: (i, 0)),
            pl.BlockSpec((8, h), lambda i: (0, 0)),
        ],
        out_shape=[
            jax.ShapeDtypeStruct((N, h), jnp.float32),
            jax.ShapeDtypeStruct((8, h), jnp.float32),
        ],
    )(p, qg, b1, w2, b2)


def _heada_body(s1_ref, s2_ref, s3_ref, st1_ref, st2_ref, st3_ref,
                g1_ref, be1_ref, g2_ref, be2_ref, g3_ref, be3_ref,
                l1w_ref, l1b_ref, l2w_ref, l2b_ref, bf_ref,
                hl_ref, gm_ref):
    i = pl.program_id(0)

    def norm(s_ref, st_ref, g_ref, be_ref):
        mu = st_ref[0:1, :] / N
        var = st_ref[1:2, :] / N - mu * mu
        inv = lax.rsqrt(var + 1e-5)
        return (s_ref[...] - mu) * inv * g_ref[...] + be_ref[...]

    h1 = norm(s1_ref, st1_ref, g1_ref, be1_ref)
    h2 = norm(s2_ref, st2_ref, g2_ref, be2_ref)
    h3 = norm(s3_ref, st3_ref, g3_ref, be3_ref)
    acc = (jnp.dot(h1, l1w_ref[0:96, :], preferred_element_type=jnp.float32)
           + jnp.dot(h2, l1w_ref[96:256, :], preferred_element_type=jnp.float32)
           + jnp.dot(h3, l1w_ref[256:512, :], preferred_element_type=jnp.float32)
           + l1b_ref[...])
    hl = jax.nn.relu(acc)
    hl2 = jax.nn.relu(jnp.dot(hl, l2w_ref[...],
                              preferred_element_type=jnp.float32) + l2b_ref[...])
    hl_ref[...] = hl2

    @pl.when(i == 0)
    def _():
        rows = lax.broadcasted_iota(jnp.int32, (8, 256), 0)
        gm_ref[...] = jnp.where(rows < B, -1e30, 0.0)

    bf = bf_ref[...]
    for b in range(B):
        cand = jnp.max(jnp.where(bf == float(b), hl2, -1e30),
                       axis=0, keepdims=True)
        gm_ref[b:b + 1, :] = jnp.maximum(gm_ref[b:b + 1, :], cand)


def _heada(s1, s2, s3, st1, st2, st3, g1, be1, g2, be2, g3, be3,
           l1w, l1b, l2w, l2b, batch_f):
    bf = batch_f[:, None]
    full = lambda a: pl.BlockSpec(a.shape, lambda i: (0, 0))
    return pl.pallas_call(
        _heada_body,
        grid=(N // BN_,),
        in_specs=[
            pl.BlockSpec((BN_, 96), lambda i: (i, 0)),
            pl.BlockSpec((BN_, 160), lambda i: (i, 0)),
            pl.BlockSpec((BN_, 256), lambda i: (i, 0)),
            full(st1), full(st2), full(st3),
            full(g1), full(be1), full(g2), full(be2), full(g3), full(be3),
            full(l1w), full(l1b), full(l2w), full(l2b),
            pl.BlockSpec((BN_, 1), lambda i: (i, 0)),
        ],
        out_specs=[
            pl.BlockSpec((BN_, 256), lambda i: (i, 0)),
            pl.BlockSpec((8, 256), lambda i: (0, 0)),
        ],
        out_shape=[
            jax.ShapeDtypeStruct((N, 256), jnp.float32),
            jax.ShapeDtypeStruct((8, 256), jnp.float32),
        ],
    )(s1, s2, s3, st1, st2, st3, g1, be1, g2, be2, g3, be3,
      l1w, l1b, l2w, l2b, bf)


def _headb_body(gm_ref, pg_ref, gw_ref, gb_ref, pw_ref, pb_ref,
                lng_ref, lnb_ref, h1w_ref, h1b_ref, cg_ref):
    gg = jax.nn.relu(jnp.dot(gm_ref[...], gw_ref[...],
                             preferred_element_type=jnp.float32) + gb_ref[...])
    z = jnp.dot(pg_ref[...], pw_ref[...],
                preferred_element_type=jnp.float32) + pb_ref[...]
    mu = jnp.mean(z, axis=-1, keepdims=True)
    var = jnp.mean(z * z, axis=-1, keepdims=True) - mu * mu
    zn = (z - mu) * lax.rsqrt(var + 1e-5) * lng_ref[...] + lnb_ref[...]
    gs = zn * (1.0 / (1.0 + jnp.exp(-zn)))
    cg_ref[...] = (jnp.dot(gg, h1w_ref[256:512, :],
                           preferred_element_type=jnp.float32)
                   + jnp.dot(gs, h1w_ref[512:768, :],
                             preferred_element_type=jnp.float32)
                   + h1b_ref[...])


def _headb(gm, pg8, gw, gb, pw, pb, lng, lnb, h1w, h1b):
    full = lambda a: pl.BlockSpec(a.shape, lambda: (0, 0))
    return pl.pallas_call(
        _headb_body,
        in_specs=[full(gm), full(pg8), full(gw), full(gb), full(pw), full(pb),
                  full(lng), full(lnb), full(h1w), full(h1b)],
        out_specs=full(jnp.zeros((8, 256))),
        out_shape=jax.ShapeDtypeStruct((8, 256), jnp.float32),
    )(gm, pg8, gw, gb, pw, pb, lng, lnb, h1w, h1b)


def _headd_body(hl_ref, cg_ref, h1w_ref, h2w_ref, h2b_ref, bf_ref, out_ref):
    bf = bf_ref[...]
    sel = jnp.zeros((BN_, 256), jnp.float32)
    for b in range(B):
        sel = sel + jnp.where(bf == float(b), cg_ref[b:b + 1, :], 0.0)
    h4 = jax.nn.relu(jnp.dot(hl_ref[...], h1w_ref[0:256, :],
                             preferred_element_type=jnp.float32) + sel)
    out_ref[...] = jnp.dot(h4, h2w_ref[...],
                           preferred_element_type=jnp.float32) + h2b_ref[...]


def _headd(hl, cg, h1w, h2wp, h2bp, batch_f):
    bf = batch_f[:, None]
    full = lambda a: pl.BlockSpec(a.shape, lambda i: (0, 0))
    return pl.pallas_call(
        _headd_body,
        grid=(N // BN_,),
        in_specs=[
            pl.BlockSpec((BN_, 256), lambda i: (i, 0)),
            full(cg), full(h1w), full(h2wp), full(h2bp),
            pl.BlockSpec((BN_, 1), lambda i: (i, 0)),
        ],
        out_specs=pl.BlockSpec((BN_, 8), lambda i: (i, 0)),
        out_shape=jax.ShapeDtypeStruct((N, 8), jnp.float32),
    )(hl, cg, h1w, h2wp, h2bp, bf)


def kernel(xyz, x, pos_in, batch, pretrain_global, params):
    p = params
    batch_f = batch.astype(jnp.float32)

    col = _knn(pos_in, batch_f)
    col_pad = jnp.pad(col.reshape(E), (0, EPAD - E))

    def padc(w, hp):
        return jnp.pad(w, ((0, 0), (0, hp - w.shape[1])))

    def padr(w, hp):
        return jnp.pad(w, ((0, hp - w.shape[0]), (0, 0)))

    xin = jnp.pad(jnp.concatenate([xyz, x], axis=1), ((0, 0), (0, 6)))
    w0p = jnp.pad(p['W0'], ((0, 6), (0, 0)))
    p1, q1 = _pq1(xin, w0p, p['b0'][None, :], padc(p['e1w1'], 128), 128)
    qg1 = _sc_gather(q1, col_pad, 128)
    s1, st1 = _edge(p1, qg1, padc(p['e1b1'][None, :], 128),
                    padr(p['e1w2'], 128), p['e1b2'][None, :], 96, 128)

    p2, q2 = _pq23(s1, st1, p['e1g'][None, :], p['e1be'][None, :],
                   padc(p['e2w1'], 256), 256)
    qg2 = _sc_gather(q2, col_pad, 256)
    s2, st2 = _edge(p2, qg2, padc(p['e2b1'][None, :], 256),
                    padr(p['e2w2'], 256), p['e2b2'][None, :], 160, 256)

    p3, q3 = _pq23(s2, st2, p['e2g'][None, :], p['e2be'][None, :], p['e3w1'], 256)
    qg3 = _sc_gather(q3, col_pad, 256)
    s3, st3 = _edge(p3, qg3, p['e3b1'][None, :], p['e3w2'], p['e3b2'][None, :],
                    256, 256)

    hl, gm = _heada(s1, s2, s3, st1, st2, st3,
                    p['e1g'][None, :], p['e1be'][None, :],
                    p['e2g'][None, :], p['e2be'][None, :],
                    p['e3g'][None, :], p['e3be'][None, :],
                    p['l1w'], p['l1b'][None, :], p['l2w'], p['l2b'][None, :],
                    batch_f)
    pg8 = jnp.pad(pretrain_global, ((0, 4), (0, 0)))
    cg = _headb(gm, pg8, p['gw'], p['gb'][None, :], p['pw'], p['pb'][None, :],
                p['lng'][None, :], p['lnb'][None, :], p['h1w'], p['h1b'][None, :])
    h2wp = jnp.pad(p['h2w'], ((0, 0), (0, 7)))
    h2bp = jnp.pad(p['h2b'][None, :], ((0, 0), (0, 7)))
    out = _headd(hl, cg, p['h1w'], h2wp, h2bp, batch_f)
    return out[:, 0]


# speedup vs baseline: 4.8402x; 1.0577x over previous
import functools

import jax
import jax.numpy as jnp
from jax import lax
from jax.experimental import pallas as pl
from jax.experimental.pallas import tpu as pltpu
from jax.experimental.pallas import tpu_sc as plsc

N = 10000
B = 4
K = 20
NPAD = 10240
BQ = 80
BN_ = 80
EBLK = BN_ * K
E = N * K
EPAD = 204800
BIG1 = 1e30
BIG2 = 1e31
NBIG = 1 << 30

SC_NC = 2
SC_NS = 16
SC_WORKERS = SC_NC * SC_NS
SC_PER_W = EPAD // SC_WORKERS
SC_CH = 80
SC_NB = 4
SC_T = SC_PER_W // SC_CH
SC_GROUPS = SC_T // SC_NB


def _knn_body(qpos_ref, post_ref, batchrow_ref, qbatch_ref, out_ref):
    i = pl.program_id(0)
    q = qpos_ref[...]
    pt = post_ref[...]
    d = ((q[:, 0:1] - pt[0:1, :]) ** 2 + (q[:, 1:2] - pt[1:2, :]) ** 2
         + (q[:, 2:3] - pt[2:3, :]) ** 2)
    lanes = lax.broadcasted_iota(jnp.int32, (BQ, NPAD), 1)
    qidx = i * BQ + lax.broadcasted_iota(jnp.int32, (BQ, 1), 0)
    valid = (batchrow_ref[...] == qbatch_ref[...]) & (lanes != qidx)
    dm = jnp.where(valid, d, BIG1)
    cols = []
    for _ in range(K):
        m = jnp.min(dm, axis=1, keepdims=True)
        cand = jnp.where(dm == m, lanes, NBIG)
        idx = jnp.min(cand, axis=1, keepdims=True)
        cols.append(idx)
        dm = jnp.where(lanes == idx, BIG2, dm)
    out_ref[...] = jnp.concatenate(cols, axis=1)


def _knn(pos_in, batch_f):
    pos8 = jnp.pad(pos_in, ((0, 0), (0, 5)))
    post = jnp.pad(pos_in, ((0, NPAD - N), (0, 5))).T
    batchrow = jnp.pad(batch_f[None, :], ((0, 0), (0, NPAD - N)),
                       constant_values=-1.0)
    qbatch = batch_f[:, None]
    return pl.pallas_call(
        _knn_body,
        grid=(N // BQ,),
        in_specs=[
            pl.BlockSpec((BQ, 8), lambda i: (i, 0)),
            pl.BlockSpec((8, NPAD), lambda i: (0, 0)),
            pl.BlockSpec((1, NPAD), lambda i: (0, 0)),
            pl.BlockSpec((BQ, 1), lambda i: (i, 0)),
        ],
        out_specs=pl.BlockSpec((BQ, K), lambda i: (i, 0)),
        out_shape=jax.ShapeDtypeStruct((N, K), jnp.int32),
    )(pos8, post, batchrow, qbatch)


def _pq1_body(xin_ref, w0_ref, b0_ref, w1_ref, p_ref, q_ref):
    x0 = jnp.dot(xin_ref[...], w0_ref[...],
                 preferred_element_type=jnp.float32) + b0_ref[...]
    d = w1_ref.shape[0] // 2
    wa = w1_ref[:d, :]
    wb = w1_ref[d:, :]
    p_ref[...] = jnp.dot(x0, wa - wb, preferred_element_type=jnp.float32)
    q_ref[...] = jnp.dot(x0, wb, preferred_element_type=jnp.float32)


def _pq1(xin, w0p, b0, w1, h):
    br = 400
    return pl.pallas_call(
        _pq1_body,
        grid=(N // br,),
        in_specs=[
            pl.BlockSpec((br, 16), lambda i: (i, 0)),
            pl.BlockSpec(w0p.shape, lambda i: (0, 0)),
            pl.BlockSpec((1, w0p.shape[1]), lambda i: (0, 0)),
            pl.BlockSpec(w1.shape, lambda i: (0, 0)),
        ],
        out_specs=[
            pl.BlockSpec((br, h), lambda i: (i, 0)),
            pl.BlockSpec((br, h), lambda i: (i, 0)),
        ],
        out_shape=[
            jax.ShapeDtypeStruct((N, h), jnp.float32),
            jax.ShapeDtypeStruct((N, h), jnp.float32),
        ],
    )(xin, w0p, b0, w1)


def _pq23_body(s_ref, st_ref, g_ref, be_ref, w1_ref, p_ref, q_ref):
    mu = st_ref[0:1, :] / N
    var = st_ref[1:2, :] / N - mu * mu
    inv = lax.rsqrt(var + 1e-5)
    xn = (s_ref[...] - mu) * inv * g_ref[...] + be_ref[...]
    d = w1_ref.shape[0] // 2
    wa = w1_ref[:d, :]
    wb = w1_ref[d:, :]
    p_ref[...] = jnp.dot(xn, wa - wb, preferred_element_type=jnp.float32)
    q_ref[...] = jnp.dot(xn, wb, preferred_element_type=jnp.float32)


def _pq23(s, stats, g, be, w1, h):
    br = 400
    din = s.shape[1]
    return pl.pallas_call(
        _pq23_body,
        grid=(N // br,),
        in_specs=[
            pl.BlockSpec((br, din), lambda i: (i, 0)),
            pl.BlockSpec((8, din), lambda i: (0, 0)),
            pl.BlockSpec((1, din), lambda i: (0, 0)),
            pl.BlockSpec((1, din), lambda i: (0, 0)),
            pl.BlockSpec(w1.shape, lambda i: (0, 0)),
        ],
        out_specs=[
            pl.BlockSpec((br, h), lambda i: (i, 0)),
            pl.BlockSpec((br, h), lambda i: (i, 0)),
        ],
        out_shape=[
            jax.ShapeDtypeStruct((N, h), jnp.float32),
            jax.ShapeDtypeStruct((N, h), jnp.float32),
        ],
    )(s, stats, g, be, w1)


def _sc_gather(table, idx_pad, h):
    mesh = plsc.VectorSubcoreMesh(core_axis_name="c", subcore_axis_name="s",
                                  num_cores=SC_NC, num_subcores=SC_NS)

    @functools.partial(
        pl.kernel,
        out_type=jax.ShapeDtypeStruct((EPAD, h), jnp.float32),
        mesh=mesh,
        scratch_types=(
            [pltpu.VMEM((SC_PER_W,), jnp.int32)]
            + [pltpu.VMEM((SC_CH, h), jnp.float32)] * SC_NB
            + [pltpu.SemaphoreType.DMA] * (2 * SC_NB)
        ),
    )
    def k(table_hbm, idx_hbm, out_hbm, idxall,
          r0, r1, r2, r3, g0, g1, g2, g3, o0, o1, o2, o3):
        rows = [r0, r1, r2, r3]
        gsem = [g0, g1, g2, g3]
        osem = [o0, o1, o2, o3]
        wid = lax.axis_index("s") * SC_NC + lax.axis_index("c")
        base = wid * SC_PER_W
        pltpu.sync_copy(idx_hbm.at[pl.ds(base, SC_PER_W)], idxall)

        def start_gather(t, b):
            pltpu.async_copy(
                table_hbm.at[idxall.at[pl.ds(t * SC_CH, SC_CH)]],
                rows[b], gsem[b])

        def wait_gather(b):
            pltpu.make_async_copy(
                table_hbm.at[pl.ds(0, SC_CH)], rows[b], gsem[b]).wait()

        def drain_chunk(t, b):
            wait_gather(b)
            out_slice = out_hbm.at[pl.ds(base + t * SC_CH, SC_CH)]
            pltpu.async_copy(rows[b], out_slice, osem[b])
            pltpu.make_async_copy(
                rows[b], out_hbm.at[pl.ds(base, SC_CH)], osem[b]).wait()

        for b in range(SC_NB):
            start_gather(b, b)

        def group(g, carry):
            for b in range(SC_NB):
                t = g * SC_NB + b
                drain_chunk(t, b)
                start_gather(t + SC_NB, b)
            return carry

        lax.fori_loop(0, SC_GROUPS - 1, group, 0)
        for b in range(SC_NB):
            drain_chunk((SC_GROUPS - 1) * SC_NB + b, b)

    return k(table, idx_pad)


def _edge_body(p_ref, qg_ref, b1_ref, w2_ref, b2_ref, s_ref, st_ref):
    i = pl.program_id(0)
    hpad = p_ref.shape[1]
    h = w2_ref.shape[1]
    p = p_ref[...]
    qg = qg_ref[...].reshape(BN_, K, hpad)
    a = jax.nn.relu(qg + p[:, None, :] + b1_ref[...][None])
    m = jnp.dot(a.reshape(EBLK, hpad), w2_ref[...],
                preferred_element_type=jnp.float32)
    s = jax.nn.relu(jnp.max(m.reshape(BN_, K, h), axis=1) + b2_ref[...])
    s_ref[...] = s

    @pl.when(i == 0)
    def _():
        st_ref[...] = jnp.zeros_like(st_ref)

    st_ref[0:1, :] += jnp.sum(s, axis=0, keepdims=True)
    st_ref[1:2, :] += jnp.sum(s * s, axis=0, keepdims=True)


def _edge(p, qg, b1, w2, b2, h, hpad):
    return pl.pallas_call(
        _edge_body,
        grid=(N // BN_,),
        in_specs=[
            pl.BlockSpec((BN_, hpad), lambda i: (i, 0)),
            pl.BlockSpec((EBLK, hpad), lambda i: (i, 0)),
            pl.BlockSpec((1, hpad), lambda i: (0, 0)),
            pl.BlockSpec((hpad, h), lambda i: (0, 0)),
            pl.BlockSpec((1, h), lambda i: (0, 0)),
        ],
        out_specs=[
            pl.BlockSpec((BN_, h), lambda i: (i, 0)),
            pl.BlockSpec((8, h), lambda i: (0, 0)),
        ],
        out_shape=[
            jax.ShapeDtypeStruct((N, h), jnp.float32),
            jax.ShapeDtypeStruct((8, h), jnp.float32),
        ],
    )(p, qg, b1, w2, b2)


def _heada_body(s1_ref, s2_ref, s3_ref, st1_ref, st2_ref, st3_ref,
                g1_ref, be1_ref, g2_ref, be2_ref, g3_ref, be3_ref,
                l1w_ref, l1b_ref, l2w_ref, l2b_ref, bf_ref,
                hl_ref, gm_ref):
    i = pl.program_id(0)

    def norm(s_ref, st_ref, g_ref, be_ref):
        mu = st_ref[0:1, :] / N
        var = st_ref[1:2, :] / N - mu * mu
        inv = lax.rsqrt(var + 1e-5)
        return (s_ref[...] - mu) * inv * g_ref[...] + be_ref[...]

    h1 = norm(s1_ref, st1_ref, g1_ref, be1_ref)
    h2 = norm(s2_ref, st2_ref, g2_ref, be2_ref)
    h3 = norm(s3_ref, st3_ref, g3_ref, be3_ref)
    acc = (jnp.dot(h1, l1w_ref[0:96, :], preferred_element_type=jnp.float32)
           + jnp.dot(h2, l1w_ref[96:256, :], preferred_element_type=jnp.float32)
           + jnp.dot(h3, l1w_ref[256:512, :], preferred_element_type=jnp.float32)
           + l1b_ref[...])
    hl = jax.nn.relu(acc)
    hl2 = jax.nn.relu(jnp.dot(hl, l2w_ref[...],
                              preferred_element_type=jnp.float32) + l2b_ref[...])
    hl_ref[...] = hl2

    @pl.when(i == 0)
    def _():
        rows = lax.broadcasted_iota(jnp.int32, (8, 256), 0)
        gm_ref[...] = jnp.where(rows < B, -1e30, 0.0)

    bf = bf_ref[...]
    for b in range(B):
        cand = jnp.max(jnp.where(bf == float(b), hl2, -1e30),
                       axis=0, keepdims=True)
        gm_ref[b:b + 1, :] = jnp.maximum(gm_ref[b:b + 1, :], cand)


def _heada(s1, s2, s3, st1, st2, st3, g1, be1, g2, be2, g3, be3,
           l1w, l1b, l2w, l2b, batch_f):
    bf = batch_f[:, None]
    full = lambda a: pl.BlockSpec(a.shape, lambda i: (0, 0))
    return pl.pallas_call(
        _heada_body,
        grid=(N // BN_,),
        in_specs=[
            pl.BlockSpec((BN_, 96), lambda i: (i, 0)),
            pl.BlockSpec((BN_, 160), lambda i: (i, 0)),
            pl.BlockSpec((BN_, 256), lambda i: (i, 0)),
            full(st1), full(st2), full(st3),
            full(g1), full(be1), full(g2), full(be2), full(g3), full(be3),
            full(l1w), full(l1b), full(l2w), full(l2b),
            pl.BlockSpec((BN_, 1), lambda i: (i, 0)),
        ],
        out_specs=[
            pl.BlockSpec((BN_, 256), lambda i: (i, 0)),
            pl.BlockSpec((8, 256), lambda i: (0, 0)),
        ],
        out_shape=[
            jax.ShapeDtypeStruct((N, 256), jnp.float32),
            jax.ShapeDtypeStruct((8, 256), jnp.float32),
        ],
    )(s1, s2, s3, st1, st2, st3, g1, be1, g2, be2, g3, be3,
      l1w, l1b, l2w, l2b, bf)


def _headb_body(gm_ref, pg_ref, gw_ref, gb_ref, pw_ref, pb_ref,
                lng_ref, lnb_ref, h1w_ref, h1b_ref, cg_ref):
    gg = jax.nn.relu(jnp.dot(gm_ref[...], gw_ref[...],
                             preferred_element_type=jnp.float32) + gb_ref[...])
    z = jnp.dot(pg_ref[...], pw_ref[...],
                preferred_element_type=jnp.float32) + pb_ref[...]
    mu = jnp.mean(z, axis=-1, keepdims=True)
    var = jnp.mean(z * z, axis=-1, keepdims=True) - mu * mu
    zn = (z - mu) * lax.rsqrt(var + 1e-5) * lng_ref[...] + lnb_ref[...]
    gs = zn * (1.0 / (1.0 + jnp.exp(-zn)))
    cg_ref[...] = (jnp.dot(gg, h1w_ref[256:512, :],
                           preferred_element_type=jnp.float32)
                   + jnp.dot(gs, h1w_ref[512:768, :],
                             preferred_element_type=jnp.float32)
                   + h1b_ref[...])


def _headb(gm, pg8, gw, gb, pw, pb, lng, lnb, h1w, h1b):
    full = lambda a: pl.BlockSpec(a.shape, lambda: (0, 0))
    return pl.pallas_call(
        _headb_body,
        in_specs=[full(gm), full(pg8), full(gw), full(gb), full(pw), full(pb),
                  full(lng), full(lnb), full(h1w), full(h1b)],
        out_specs=full(jnp.zeros((8, 256))),
        out_shape=jax.ShapeDtypeStruct((8, 256), jnp.float32),
    )(gm, pg8, gw, gb, pw, pb, lng, lnb, h1w, h1b)


def _headd_body(hl_ref, cg_ref, h1w_ref, h2w_ref, h2b_ref, bf_ref, out_ref):
    bf = bf_ref[...]
    sel = jnp.zeros((BN_, 256), jnp.float32)
    for b in range(B):
        sel = sel + jnp.where(bf == float(b), cg_ref[b:b + 1, :], 0.0)
    h4 = jax.nn.relu(jnp.dot(hl_ref[...], h1w_ref[0:256, :],
                             preferred_element_type=jnp.float32) + sel)
    out_ref[...] = jnp.dot(h4, h2w_ref[...],
                           preferred_element_type=jnp.float32) + h2b_ref[...]


def _headd(hl, cg, h1w, h2wp, h2bp, batch_f):
    bf = batch_f[:, None]
    full = lambda a: pl.BlockSpec(a.shape, lambda i: (0, 0))
    return pl.pallas_call(
        _headd_body,
        grid=(N // BN_,),
        in_specs=[
            pl.BlockSpec((BN_, 256), lambda i: (i, 0)),
            full(cg), full(h1w), full(h2wp), full(h2bp),
            pl.BlockSpec((BN_, 1), lambda i: (i, 0)),
        ],
        out_specs=pl.BlockSpec((BN_, 8), lambda i: (i, 0)),
        out_shape=jax.ShapeDtypeStruct((N, 8), jnp.float32),
    )(hl, cg, h1w, h2wp, h2bp, bf)


def kernel(xyz, x, pos_in, batch, pretrain_global, params):
    p = params
    batch_f = batch.astype(jnp.float32)

    col = _knn(pos_in, batch_f)
    col_pad = jnp.pad(col.reshape(E), (0, EPAD - E))

    def padc(w, hp):
        return jnp.pad(w, ((0, 0), (0, hp - w.shape[1])))

    def padr(w, hp):
        return jnp.pad(w, ((0, hp - w.shape[0]), (0, 0)))

    xin = jnp.pad(jnp.concatenate([xyz, x], axis=1), ((0, 0), (0, 6)))
    w0p = jnp.pad(p['W0'], ((0, 6), (0, 0)))
    p1, q1 = _pq1(xin, w0p, p['b0'][None, :], padc(p['e1w1'], 128), 128)
    qg1 = _sc_gather(q1, col_pad, 128)
    s1, st1 = _edge(p1, qg1, padc(p['e1b1'][None, :], 128),
                    padr(p['e1w2'], 128), p['e1b2'][None, :], 96, 128)

    p2, q2 = _pq23(s1, st1, p['e1g'][None, :], p['e1be'][None, :],
                   padc(p['e2w1'], 256), 256)
    qg2 = _sc_gather(q2, col_pad, 256)
    s2, st2 = _edge(p2, qg2, padc(p['e2b1'][None, :], 256),
                    padr(p['e2w2'], 256), p['e2b2'][None, :], 160, 256)

    p3, q3 = _pq23(s2, st2, p['e2g'][None, :], p['e2be'][None, :], p['e3w1'], 256)
    qg3 = _sc_gather(q3, col_pad, 256)
    s3, st3 = _edge(p3, qg3, p['e3b1'][None, :], p['e3w2'], p['e3b2'][None, :],
                    256, 256)

    hl, gm = _heada(s1, s2, s3, st1, st2, st3,
                    p['e1g'][None, :], p['e1be'][None, :],
                    p['e2g'][None, :], p['e2be'][None, :],
                    p['e3g'][None, :], p['e3be'][None, :],
                    p['l1w'], p['l1b'][None, :], p['l2w'], p['l2b'][None, :],
                    batch_f)
    pg8 = jnp.pad(pretrain_global, ((0, 4), (0, 0)))
    cg = _headb(gm, pg8, p['gw'], p['gb'][None, :], p['pw'], p['pb'][None, :],
                p['lng'][None, :], p['lnb'][None, :], p['h1w'], p['h1b'][None, :])
    h2wp = jnp.pad(p['h2w'], ((0, 0), (0, 7)))
    h2bp = jnp.pad(p['h2b'][None, :], ((0, 0), (0, 7)))
    out = _headd(hl, cg, p['h1w'], h2wp, h2bp, batch_f)
    return out[:, 0]


# speedup vs baseline: 6.5856x; 1.3606x over previous
import functools

import jax
import jax.numpy as jnp
from jax import lax
from jax.experimental import pallas as pl
from jax.experimental.pallas import tpu as pltpu
from jax.experimental.pallas import tpu_sc as plsc

N = 10000
B = 4
K = 20
NPAD = 10240
BQ = 80
BN_ = 80
EBLK = BN_ * K
E = N * K
EPAD = 204800
BIG1 = 1e30
BIG2 = 1e31
NBIG = 1 << 30

SC_NC = 2
SC_NS = 16
SC_WORKERS = SC_NC * SC_NS
SC_PER_W = EPAD // SC_WORKERS
SC_CH = 80
SC_NB = 4
SC_T = SC_PER_W // SC_CH
SC_GROUPS = SC_T // SC_NB


WKNN = 1280
JWIN = NPAD // WKNN
STATEW = 32
BIG3 = 1e32


def _knn_body(wref, nref, qpos_ref, post_ref, batchrow_ref, qbatch_ref,
              out_ref, curv_ref, curi_ref):
    i = pl.program_id(0)
    j = pl.program_id(1)

    @pl.when(j == 0)
    def _():
        curv_ref[...] = jnp.full((BQ, STATEW), BIG3, jnp.float32)
        curi_ref[...] = jnp.full((BQ, STATEW), NBIG, jnp.int32)

    @pl.when(j < nref[i])
    def _():
        q = qpos_ref[...]
        pt = post_ref[...]
        d = ((q[:, 0:1] - pt[0:1, :]) ** 2 + (q[:, 1:2] - pt[1:2, :]) ** 2
             + (q[:, 2:3] - pt[2:3, :]) ** 2)
        gcol = ((wref[i] + j) * WKNN
                + lax.broadcasted_iota(jnp.int32, (BQ, WKNN), 1))
        qidx = i * BQ + lax.broadcasted_iota(jnp.int32, (BQ, 1), 0)
        valid = (batchrow_ref[...] == qbatch_ref[...]) & (gcol != qidx)
        dm = jnp.where(valid, d, BIG1)
        catv = jnp.concatenate([dm, curv_ref[...]], axis=1)
        cati = jnp.concatenate([gcol, curi_ref[...]], axis=1)
        vals, idxs = [], []
        for _ in range(K):
            m = jnp.min(catv, axis=1, keepdims=True)
            idx = jnp.min(jnp.where(catv == m, cati, NBIG),
                          axis=1, keepdims=True)
            vals.append(m)
            idxs.append(idx)
            catv = jnp.where(cati == idx, BIG2, catv)
        pad_v = jnp.full((BQ, STATEW - K), BIG3, jnp.float32)
        pad_i = jnp.full((BQ, STATEW - K), NBIG, jnp.int32)
        curv_ref[...] = jnp.concatenate(vals + [pad_v], axis=1)
        curi_ref[...] = jnp.concatenate(idxs + [pad_i], axis=1)

        @pl.when(j == nref[i] - 1)
        def _():
            out_ref[...] = jnp.concatenate(idxs, axis=1)


def _knn(pos_in, batch, batch_f):
    pos8 = jnp.pad(pos_in, ((0, 0), (0, 5)))
    post = jnp.pad(pos_in, ((0, NPAD - N), (0, 5))).T
    batchrow = jnp.pad(batch_f[None, :], ((0, 0), (0, NPAD - N)),
                       constant_values=-1.0)
    qbatch = batch_f[:, None]
    gfirst = jnp.searchsorted(batch, jnp.arange(B), side='left')
    glast = jnp.searchsorted(batch, jnp.arange(B), side='right')
    gs = gfirst[batch[::BQ]]
    ge = glast[batch[BQ - 1::BQ]]
    wstart = (gs // WKNN).astype(jnp.int32)
    nwin = ((ge - 1) // WKNN - gs // WKNN + 1).astype(jnp.int32)
    grid_spec = pltpu.PrefetchScalarGridSpec(
        num_scalar_prefetch=2,
        grid=(N // BQ, JWIN),
        in_specs=[
            pl.BlockSpec((BQ, 8), lambda i, j, w, nw: (i, 0)),
            pl.BlockSpec((8, WKNN),
                         lambda i, j, w, nw: (0, jnp.minimum(w[i] + j, JWIN - 1))),
            pl.BlockSpec((1, WKNN),
                         lambda i, j, w, nw: (0, jnp.minimum(w[i] + j, JWIN - 1))),
            pl.BlockSpec((BQ, 1), lambda i, j, w, nw: (i, 0)),
        ],
        out_specs=pl.BlockSpec((BQ, K), lambda i, j, w, nw: (i, 0)),
        scratch_shapes=[
            pltpu.VMEM((BQ, STATEW), jnp.float32),
            pltpu.VMEM((BQ, STATEW), jnp.int32),
        ],
    )
    return pl.pallas_call(
        _knn_body,
        grid_spec=grid_spec,
        out_shape=jax.ShapeDtypeStruct((N, K), jnp.int32),
    )(wstart, nwin, pos8, post, batchrow, qbatch)


def _pq1_body(xin_ref, w0_ref, b0_ref, w1_ref, p_ref, q_ref):
    x0 = jnp.dot(xin_ref[...], w0_ref[...],
                 preferred_element_type=jnp.float32) + b0_ref[...]
    d = w1_ref.shape[0] // 2
    wa = w1_ref[:d, :]
    wb = w1_ref[d:, :]
    p_ref[...] = jnp.dot(x0, wa - wb, preferred_element_type=jnp.float32)
    q_ref[...] = jnp.dot(x0, wb, preferred_element_type=jnp.float32)


def _pq1(xin, w0p, b0, w1, h):
    br = 400
    return pl.pallas_call(
        _pq1_body,
        grid=(N // br,),
        in_specs=[
            pl.BlockSpec((br, 16), lambda i: (i, 0)),
            pl.BlockSpec(w0p.shape, lambda i: (0, 0)),
            pl.BlockSpec((1, w0p.shape[1]), lambda i: (0, 0)),
            pl.BlockSpec(w1.shape, lambda i: (0, 0)),
        ],
        out_specs=[
            pl.BlockSpec((br, h), lambda i: (i, 0)),
            pl.BlockSpec((br, h), lambda i: (i, 0)),
        ],
        out_shape=[
            jax.ShapeDtypeStruct((N, h), jnp.float32),
            jax.ShapeDtypeStruct((N, h), jnp.float32),
        ],
    )(xin, w0p, b0, w1)


def _pq23_body(s_ref, st_ref, g_ref, be_ref, w1_ref, p_ref, q_ref):
    mu = st_ref[0:1, :] / N
    var = st_ref[1:2, :] / N - mu * mu
    inv = lax.rsqrt(var + 1e-5)
    xn = (s_ref[...] - mu) * inv * g_ref[...] + be_ref[...]
    d = w1_ref.shape[0] // 2
    wa = w1_ref[:d, :]
    wb = w1_ref[d:, :]
    p_ref[...] = jnp.dot(xn, wa - wb, preferred_element_type=jnp.float32)
    q_ref[...] = jnp.dot(xn, wb, preferred_element_type=jnp.float32)


def _pq23(s, stats, g, be, w1, h):
    br = 400
    din = s.shape[1]
    return pl.pallas_call(
        _pq23_body,
        grid=(N // br,),
        in_specs=[
            pl.BlockSpec((br, din), lambda i: (i, 0)),
            pl.BlockSpec((8, din), lambda i: (0, 0)),
            pl.BlockSpec((1, din), lambda i: (0, 0)),
            pl.BlockSpec((1, din), lambda i: (0, 0)),
            pl.BlockSpec(w1.shape, lambda i: (0, 0)),
        ],
        out_specs=[
            pl.BlockSpec((br, h), lambda i: (i, 0)),
            pl.BlockSpec((br, h), lambda i: (i, 0)),
        ],
        out_shape=[
            jax.ShapeDtypeStruct((N, h), jnp.float32),
            jax.ShapeDtypeStruct((N, h), jnp.float32),
        ],
    )(s, stats, g, be, w1)


def _sc_gather(table, idx_pad, h):
    mesh = plsc.VectorSubcoreMesh(core_axis_name="c", subcore_axis_name="s",
                                  num_cores=SC_NC, num_subcores=SC_NS)

    @functools.partial(
        pl.kernel,
        out_type=jax.ShapeDtypeStruct((EPAD, h), jnp.float32),
        mesh=mesh,
        scratch_types=(
            [pltpu.VMEM((SC_PER_W,), jnp.int32)]
            + [pltpu.VMEM((SC_CH, h), jnp.float32)] * SC_NB
            + [pltpu.SemaphoreType.DMA] * (2 * SC_NB)
        ),
    )
    def k(table_hbm, idx_hbm, out_hbm, idxall,
          r0, r1, r2, r3, g0, g1, g2, g3, o0, o1, o2, o3):
        rows = [r0, r1, r2, r3]
        gsem = [g0, g1, g2, g3]
        osem = [o0, o1, o2, o3]
        wid = lax.axis_index("s") * SC_NC + lax.axis_index("c")
        base = wid * SC_PER_W
        pltpu.sync_copy(idx_hbm.at[pl.ds(base, SC_PER_W)], idxall)

        def start_gather(t, b):
            pltpu.async_copy(
                table_hbm.at[idxall.at[pl.ds(t * SC_CH, SC_CH)]],
                rows[b], gsem[b])

        def wait_gather(b):
            pltpu.make_async_copy(
                table_hbm.at[pl.ds(0, SC_CH)], rows[b], gsem[b]).wait()

        def drain_chunk(t, b):
            wait_gather(b)
            out_slice = out_hbm.at[pl.ds(base + t * SC_CH, SC_CH)]
            pltpu.async_copy(rows[b], out_slice, osem[b])
            pltpu.make_async_copy(
                rows[b], out_hbm.at[pl.ds(base, SC_CH)], osem[b]).wait()

        for b in range(SC_NB):
            start_gather(b, b)

        def group(g, carry):
            for b in range(SC_NB):
                t = g * SC_NB + b
                drain_chunk(t, b)
                start_gather(t + SC_NB, b)
            return carry

        lax.fori_loop(0, SC_GROUPS - 1, group, 0)
        for b in range(SC_NB):
            drain_chunk((SC_GROUPS - 1) * SC_NB + b, b)

    return k(table, idx_pad)


def _edge_body(p_ref, qg_ref, b1_ref, w2_ref, b2_ref, s_ref, st_ref):
    i = pl.program_id(0)
    hpad = p_ref.shape[1]
    h = w2_ref.shape[1]
    p = p_ref[...]
    qg = qg_ref[...].reshape(BN_, K, hpad)
    a = jax.nn.relu(qg + p[:, None, :] + b1_ref[...][None])
    m = jnp.dot(a.reshape(EBLK, hpad), w2_ref[...],
                preferred_element_type=jnp.float32)
    s = jax.nn.relu(jnp.max(m.reshape(BN_, K, h), axis=1) + b2_ref[...])
    s_ref[...] = s

    @pl.when(i == 0)
    def _():
        st_ref[...] = jnp.zeros_like(st_ref)

    st_ref[0:1, :] += jnp.sum(s, axis=0, keepdims=True)
    st_ref[1:2, :] += jnp.sum(s * s, axis=0, keepdims=True)


def _edge(p, qg, b1, w2, b2, h, hpad):
    return pl.pallas_call(
        _edge_body,
        grid=(N // BN_,),
        in_specs=[
            pl.BlockSpec((BN_, hpad), lambda i: (i, 0)),
            pl.BlockSpec((EBLK, hpad), lambda i: (i, 0)),
            pl.BlockSpec((1, hpad), lambda i: (0, 0)),
            pl.BlockSpec((hpad, h), lambda i: (0, 0)),
            pl.BlockSpec((1, h), lambda i: (0, 0)),
        ],
        out_specs=[
            pl.BlockSpec((BN_, h), lambda i: (i, 0)),
            pl.BlockSpec((8, h), lambda i: (0, 0)),
        ],
        out_shape=[
            jax.ShapeDtypeStruct((N, h), jnp.float32),
            jax.ShapeDtypeStruct((8, h), jnp.float32),
        ],
    )(p, qg, b1, w2, b2)


def _heada_body(s1_ref, s2_ref, s3_ref, st1_ref, st2_ref, st3_ref,
                g1_ref, be1_ref, g2_ref, be2_ref, g3_ref, be3_ref,
                l1w_ref, l1b_ref, l2w_ref, l2b_ref, bf_ref,
                hl_ref, gm_ref):
    i = pl.program_id(0)

    def norm(s_ref, st_ref, g_ref, be_ref):
        mu = st_ref[0:1, :] / N
        var = st_ref[1:2, :] / N - mu * mu
        inv = lax.rsqrt(var + 1e-5)
        return (s_ref[...] - mu) * inv * g_ref[...] + be_ref[...]

    h1 = norm(s1_ref, st1_ref, g1_ref, be1_ref)
    h2 = norm(s2_ref, st2_ref, g2_ref, be2_ref)
    h3 = norm(s3_ref, st3_ref, g3_ref, be3_ref)
    acc = (jnp.dot(h1, l1w_ref[0:96, :], preferred_element_type=jnp.float32)
           + jnp.dot(h2, l1w_ref[96:256, :], preferred_element_type=jnp.float32)
           + jnp.dot(h3, l1w_ref[256:512, :], preferred_element_type=jnp.float32)
           + l1b_ref[...])
    hl = jax.nn.relu(acc)
    hl2 = jax.nn.relu(jnp.dot(hl, l2w_ref[...],
                              preferred_element_type=jnp.float32) + l2b_ref[...])
    hl_ref[...] = hl2

    @pl.when(i == 0)
    def _():
        rows = lax.broadcasted_iota(jnp.int32, (8, 256), 0)
        gm_ref[...] = jnp.where(rows < B, -1e30, 0.0)

    bf = bf_ref[...]
    for b in range(B):
        cand = jnp.max(jnp.where(bf == float(b), hl2, -1e30),
                       axis=0, keepdims=True)
        gm_ref[b:b + 1, :] = jnp.maximum(gm_ref[b:b + 1, :], cand)


def _heada(s1, s2, s3, st1, st2, st3, g1, be1, g2, be2, g3, be3,
           l1w, l1b, l2w, l2b, batch_f):
    bf = batch_f[:, None]
    full = lambda a: pl.BlockSpec(a.shape, lambda i: (0, 0))
    return pl.pallas_call(
        _heada_body,
        grid=(N // BN_,),
        in_specs=[
            pl.BlockSpec((BN_, 96), lambda i: (i, 0)),
            pl.BlockSpec((BN_, 160), lambda i: (i, 0)),
            pl.BlockSpec((BN_, 256), lambda i: (i, 0)),
            full(st1), full(st2), full(st3),
            full(g1), full(be1), full(g2), full(be2), full(g3), full(be3),
            full(l1w), full(l1b), full(l2w), full(l2b),
            pl.BlockSpec((BN_, 1), lambda i: (i, 0)),
        ],
        out_specs=[
            pl.BlockSpec((BN_, 256), lambda i: (i, 0)),
            pl.BlockSpec((8, 256), lambda i: (0, 0)),
        ],
        out_shape=[
            jax.ShapeDtypeStruct((N, 256), jnp.float32),
            jax.ShapeDtypeStruct((8, 256), jnp.float32),
        ],
    )(s1, s2, s3, st1, st2, st3, g1, be1, g2, be2, g3, be3,
      l1w, l1b, l2w, l2b, bf)


def _headb_body(gm_ref, pg_ref, gw_ref, gb_ref, pw_ref, pb_ref,
                lng_ref, lnb_ref, h1w_ref, h1b_ref, cg_ref):
    gg = jax.nn.relu(jnp.dot(gm_ref[...], gw_ref[...],
                             preferred_element_type=jnp.float32) + gb_ref[...])
    z = jnp.dot(pg_ref[...], pw_ref[...],
                preferred_element_type=jnp.float32) + pb_ref[...]
    mu = jnp.mean(z, axis=-1, keepdims=True)
    var = jnp.mean(z * z, axis=-1, keepdims=True) - mu * mu
    zn = (z - mu) * lax.rsqrt(var + 1e-5) * lng_ref[...] + lnb_ref[...]
    gs = zn * (1.0 / (1.0 + jnp.exp(-zn)))
    cg_ref[...] = (jnp.dot(gg, h1w_ref[256:512, :],
                           preferred_element_type=jnp.float32)
                   + jnp.dot(gs, h1w_ref[512:768, :],
                             preferred_element_type=jnp.float32)
                   + h1b_ref[...])


def _headb(gm, pg8, gw, gb, pw, pb, lng, lnb, h1w, h1b):
    full = lambda a: pl.BlockSpec(a.shape, lambda: (0, 0))
    return pl.pallas_call(
        _headb_body,
        in_specs=[full(gm), full(pg8), full(gw), full(gb), full(pw), full(pb),
                  full(lng), full(lnb), full(h1w), full(h1b)],
        out_specs=full(jnp.zeros((8, 256))),
        out_shape=jax.ShapeDtypeStruct((8, 256), jnp.float32),
    )(gm, pg8, gw, gb, pw, pb, lng, lnb, h1w, h1b)


def _headd_body(hl_ref, cg_ref, h1w_ref, h2w_ref, h2b_ref, bf_ref, out_ref):
    bf = bf_ref[...]
    sel = jnp.zeros((BN_, 256), jnp.float32)
    for b in range(B):
        sel = sel + jnp.where(bf == float(b), cg_ref[b:b + 1, :], 0.0)
    h4 = jax.nn.relu(jnp.dot(hl_ref[...], h1w_ref[0:256, :],
                             preferred_element_type=jnp.float32) + sel)
    out_ref[...] = jnp.dot(h4, h2w_ref[...],
                           preferred_element_type=jnp.float32) + h2b_ref[...]


def _headd(hl, cg, h1w, h2wp, h2bp, batch_f):
    bf = batch_f[:, None]
    full = lambda a: pl.BlockSpec(a.shape, lambda i: (0, 0))
    return pl.pallas_call(
        _headd_body,
        grid=(N // BN_,),
        in_specs=[
            pl.BlockSpec((BN_, 256), lambda i: (i, 0)),
            full(cg), full(h1w), full(h2wp), full(h2bp),
            pl.BlockSpec((BN_, 1), lambda i: (i, 0)),
        ],
        out_specs=pl.BlockSpec((BN_, 8), lambda i: (i, 0)),
        out_shape=jax.ShapeDtypeStruct((N, 8), jnp.float32),
    )(hl, cg, h1w, h2wp, h2bp, bf)


def kernel(xyz, x, pos_in, batch, pretrain_global, params):
    p = params
    batch_f = batch.astype(jnp.float32)

    col = _knn(pos_in, batch, batch_f)
    tail = (jnp.arange(EPAD - E, dtype=jnp.int32) * 16) % N
    col_pad = jnp.concatenate([col.reshape(E), tail])

    def padc(w, hp):
        return jnp.pad(w, ((0, 0), (0, hp - w.shape[1])))

    def padr(w, hp):
        return jnp.pad(w, ((0, hp - w.shape[0]), (0, 0)))

    xin = jnp.pad(jnp.concatenate([xyz, x], axis=1), ((0, 0), (0, 6)))
    w0p = jnp.pad(p['W0'], ((0, 6), (0, 0)))
    p1, q1 = _pq1(xin, w0p, p['b0'][None, :], padc(p['e1w1'], 128), 128)
    qg1 = _sc_gather(q1, col_pad, 128)
    s1, st1 = _edge(p1, qg1, padc(p['e1b1'][None, :], 128),
                    padr(p['e1w2'], 128), p['e1b2'][None, :], 96, 128)

    p2, q2 = _pq23(s1, st1, p['e1g'][None, :], p['e1be'][None, :],
                   padc(p['e2w1'], 256), 256)
    qg2 = _sc_gather(q2, col_pad, 256)
    s2, st2 = _edge(p2, qg2, padc(p['e2b1'][None, :], 256),
                    padr(p['e2w2'], 256), p['e2b2'][None, :], 160, 256)

    p3, q3 = _pq23(s2, st2, p['e2g'][None, :], p['e2be'][None, :], p['e3w1'], 256)
    qg3 = _sc_gather(q3, col_pad, 256)
    s3, st3 = _edge(p3, qg3, p['e3b1'][None, :], p['e3w2'], p['e3b2'][None, :],
                    256, 256)

    hl, gm = _heada(s1, s2, s3, st1, st2, st3,
                    p['e1g'][None, :], p['e1be'][None, :],
                    p['e2g'][None, :], p['e2be'][None, :],
                    p['e3g'][None, :], p['e3be'][None, :],
                    p['l1w'], p['l1b'][None, :], p['l2w'], p['l2b'][None, :],
                    batch_f)
    pg8 = jnp.pad(pretrain_global, ((0, 4), (0, 0)))
    cg = _headb(gm, pg8, p['gw'], p['gb'][None, :], p['pw'], p['pb'][None, :],
                p['lng'][None, :], p['lnb'][None, :], p['h1w'], p['h1b'][None, :])
    h2wp = jnp.pad(p['h2w'], ((0, 0), (0, 7)))
    h2bp = jnp.pad(p['h2b'][None, :], ((0, 0), (0, 7)))
    out = _headd(hl, cg, p['h1w'], h2wp, h2bp, batch_f)
    return out[:, 0]


# speedup vs baseline: 8.6651x; 1.3158x over previous
import functools

import jax
import jax.numpy as jnp
from jax import lax
from jax.experimental import pallas as pl
from jax.experimental.pallas import tpu as pltpu
from jax.experimental.pallas import tpu_sc as plsc

N = 10000
B = 4
K = 20
NH = N // 2
NPAD = 10240
BQ = 200
BN_ = 80
BNE = 40
EBLK = BNE * K
EH = NH * K
EHPAD = 102400
BIG1 = 1e30
BIG2 = 1e31
NBIG = 1 << 30

SC_NC = 2
SC_NS = 16
SC_WORKERS = SC_NC * SC_NS
SC_PER_W = EHPAD // SC_WORKERS
SC_CH = 80
SC_NB = 4
SC_T = SC_PER_W // SC_CH
SC_GROUPS = SC_T // SC_NB


WKNN = 1280
JWIN = NPAD // WKNN
STATEW = 32
BIG3 = 1e32


def _knn_body(row0, wref, nref, qpos_ref, post_ref, batchrow_ref, qbatch_ref,
              out_ref, curv_ref, curi_ref):
    i = pl.program_id(0)
    j = pl.program_id(1)

    @pl.when(j == 0)
    def _():
        curv_ref[...] = jnp.full((BQ, STATEW), BIG3, jnp.float32)
        curi_ref[...] = jnp.full((BQ, STATEW), NBIG, jnp.int32)

    @pl.when(j < nref[i])
    def _():
        q = qpos_ref[...]
        pt = post_ref[...]
        d = ((q[:, 0:1] - pt[0:1, :]) ** 2 + (q[:, 1:2] - pt[1:2, :]) ** 2
             + (q[:, 2:3] - pt[2:3, :]) ** 2)
        gcol = ((wref[i] + j) * WKNN
                + lax.broadcasted_iota(jnp.int32, (BQ, WKNN), 1))
        qidx = row0 + i * BQ + lax.broadcasted_iota(jnp.int32, (BQ, 1), 0)
        valid = (batchrow_ref[...] == qbatch_ref[...]) & (gcol != qidx)
        dm = jnp.where(valid, d, BIG1)
        catv = jnp.concatenate([dm, curv_ref[...]], axis=1)
        cati = jnp.concatenate([gcol, curi_ref[...]], axis=1)
        vals, idxs = [], []
        for _ in range(K):
            m = jnp.min(catv, axis=1, keepdims=True)
            idx = jnp.min(jnp.where(catv == m, cati, NBIG),
                          axis=1, keepdims=True)
            vals.append(m)
            idxs.append(idx)
            catv = jnp.where(cati == idx, BIG2, catv)
        pad_v = jnp.full((BQ, STATEW - K), BIG3, jnp.float32)
        pad_i = jnp.full((BQ, STATEW - K), NBIG, jnp.int32)
        curv_ref[...] = jnp.concatenate(vals + [pad_v], axis=1)
        curi_ref[...] = jnp.concatenate(idxs + [pad_i], axis=1)

        @pl.when(j == nref[i] - 1)
        def _():
            out_ref[...] = jnp.concatenate(idxs, axis=1)


def _knn(pos_in, batch, batch_f, half):
    row0 = half * NH
    pos8 = jnp.pad(pos_in, ((0, 0), (0, 5)))
    post = jnp.pad(pos_in, ((0, NPAD - N), (0, 5))).T
    batchrow = jnp.pad(batch_f[None, :], ((0, 0), (0, NPAD - N)),
                       constant_values=-1.0)
    qbatch = batch_f[row0:row0 + NH, None]
    qpos = pos8[row0:row0 + NH]
    gfirst = jnp.searchsorted(batch, jnp.arange(B), side='left')
    glast = jnp.searchsorted(batch, jnp.arange(B), side='right')
    gs = gfirst[batch[row0:row0 + NH:BQ]]
    ge = glast[batch[row0 + BQ - 1:row0 + NH:BQ]]
    wstart = (gs // WKNN).astype(jnp.int32)
    nwin = ((ge - 1) // WKNN - gs // WKNN + 1).astype(jnp.int32)
    grid_spec = pltpu.PrefetchScalarGridSpec(
        num_scalar_prefetch=2,
        grid=(NH // BQ, JWIN),
        in_specs=[
            pl.BlockSpec((BQ, 8), lambda i, j, w, nw: (i, 0)),
            pl.BlockSpec((8, WKNN),
                         lambda i, j, w, nw: (0, jnp.minimum(w[i] + j, JWIN - 1))),
            pl.BlockSpec((1, WKNN),
                         lambda i, j, w, nw: (0, jnp.minimum(w[i] + j, JWIN - 1))),
            pl.BlockSpec((BQ, 1), lambda i, j, w, nw: (i, 0)),
        ],
        out_specs=pl.BlockSpec((BQ, K), lambda i, j, w, nw: (i, 0)),
        scratch_shapes=[
            pltpu.VMEM((BQ, STATEW), jnp.float32),
            pltpu.VMEM((BQ, STATEW), jnp.int32),
        ],
    )
    return pl.pallas_call(
        functools.partial(_knn_body, row0),
        grid_spec=grid_spec,
        out_shape=jax.ShapeDtypeStruct((NH, K), jnp.int32),
    )(wstart, nwin, qpos, post, batchrow, qbatch)


def _pq1_body(xin_ref, w0_ref, b0_ref, w1_ref, p_ref, q_ref):
    x0 = jnp.dot(xin_ref[...], w0_ref[...],
                 preferred_element_type=jnp.float32) + b0_ref[...]
    d = w1_ref.shape[0] // 2
    wa = w1_ref[:d, :]
    wb = w1_ref[d:, :]
    p_ref[...] = jnp.dot(x0, wa - wb, preferred_element_type=jnp.float32)
    q_ref[...] = jnp.dot(x0, wb, preferred_element_type=jnp.float32)


def _pq1(xin, w0p, b0, w1, h):
    br = 400
    return pl.pallas_call(
        _pq1_body,
        grid=(N // br,),
        in_specs=[
            pl.BlockSpec((br, 16), lambda i: (i, 0)),
            pl.BlockSpec(w0p.shape, lambda i: (0, 0)),
            pl.BlockSpec((1, w0p.shape[1]), lambda i: (0, 0)),
            pl.BlockSpec(w1.shape, lambda i: (0, 0)),
        ],
        out_specs=[
            pl.BlockSpec((br, h), lambda i: (i, 0)),
            pl.BlockSpec((br, h), lambda i: (i, 0)),
        ],
        out_shape=[
            jax.ShapeDtypeStruct((N, h), jnp.float32),
            jax.ShapeDtypeStruct((N, h), jnp.float32),
        ],
    )(xin, w0p, b0, w1)


def _pq23_body(s_ref, sta_ref, stb_ref, g_ref, be_ref, w1_ref, p_ref, q_ref):
    st = sta_ref[...] + stb_ref[...]
    mu = st[0:1, :] / N
    var = st[1:2, :] / N - mu * mu
    inv = lax.rsqrt(var + 1e-5)
    xn = (s_ref[...] - mu) * inv * g_ref[...] + be_ref[...]
    d = w1_ref.shape[0] // 2
    wa = w1_ref[:d, :]
    wb = w1_ref[d:, :]
    p_ref[...] = jnp.dot(xn, wa - wb, preferred_element_type=jnp.float32)
    q_ref[...] = jnp.dot(xn, wb, preferred_element_type=jnp.float32)


def _pq23(s, sta, stb, g, be, w1, h):
    br = 400
    din = s.shape[1]
    return pl.pallas_call(
        _pq23_body,
        grid=(N // br,),
        in_specs=[
            pl.BlockSpec((br, din), lambda i: (i, 0)),
            pl.BlockSpec((8, din), lambda i: (0, 0)),
            pl.BlockSpec((8, din), lambda i: (0, 0)),
            pl.BlockSpec((1, din), lambda i: (0, 0)),
            pl.BlockSpec((1, din), lambda i: (0, 0)),
            pl.BlockSpec(w1.shape, lambda i: (0, 0)),
        ],
        out_specs=[
            pl.BlockSpec((br, h), lambda i: (i, 0)),
            pl.BlockSpec((br, h), lambda i: (i, 0)),
        ],
        out_shape=[
            jax.ShapeDtypeStruct((N, h), jnp.float32),
            jax.ShapeDtypeStruct((N, h), jnp.float32),
        ],
    )(s, sta, stb, g, be, w1)


def _sc_gather(table, idx_pad, h):
    mesh = plsc.VectorSubcoreMesh(core_axis_name="c", subcore_axis_name="s",
                                  num_cores=SC_NC, num_subcores=SC_NS)

    @functools.partial(
        pl.kernel,
        out_type=jax.ShapeDtypeStruct((EHPAD, h), jnp.float32),
        mesh=mesh,
        scratch_types=(
            [pltpu.VMEM((SC_PER_W,), jnp.int32)]
            + [pltpu.VMEM((SC_CH, h), jnp.float32)] * SC_NB
            + [pltpu.SemaphoreType.DMA] * (2 * SC_NB)
        ),
    )
    def k(table_hbm, idx_hbm, out_hbm, idxall,
          r0, r1, r2, r3, g0, g1, g2, g3, o0, o1, o2, o3):
        rows = [r0, r1, r2, r3]
        gsem = [g0, g1, g2, g3]
        osem = [o0, o1, o2, o3]
        wid = lax.axis_index("s") * SC_NC + lax.axis_index("c")
        base = wid * SC_PER_W
        pltpu.sync_copy(idx_hbm.at[pl.ds(base, SC_PER_W)], idxall)

        def start_gather(t, b):
            pltpu.async_copy(
                table_hbm.at[idxall.at[pl.ds(t * SC_CH, SC_CH)]],
                rows[b], gsem[b])

        def wait_gather(b):
            pltpu.make_async_copy(
                table_hbm.at[pl.ds(0, SC_CH)], rows[b], gsem[b]).wait()

        def drain_chunk(t, b):
            wait_gather(b)
            out_slice = out_hbm.at[pl.ds(base + t * SC_CH, SC_CH)]
            pltpu.async_copy(rows[b], out_slice, osem[b])
            pltpu.make_async_copy(
                rows[b], out_hbm.at[pl.ds(base, SC_CH)], osem[b]).wait()

        for b in range(SC_NB):
            start_gather(b, b)

        def group(g, carry):
            for b in range(SC_NB):
                t = g * SC_NB + b
                drain_chunk(t, b)
                start_gather(t + SC_NB, b)
            return carry

        lax.fori_loop(0, SC_GROUPS - 1, group, 0)
        for b in range(SC_NB):
            drain_chunk((SC_GROUPS - 1) * SC_NB + b, b)

    return k(table, idx_pad)


def _edge_body(p_ref, qg_ref, b1_ref, w2_ref, b2_ref, s_ref, st_ref):
    i = pl.program_id(0)
    hpad = p_ref.shape[1]
    h = w2_ref.shape[1]
    p = p_ref[...]
    qg = qg_ref[...].reshape(BNE, K, hpad)
    a = jax.nn.relu(qg + p[:, None, :] + b1_ref[...][None])
    m = jnp.dot(a.reshape(EBLK, hpad), w2_ref[...],
                preferred_element_type=jnp.float32)
    s = jax.nn.relu(jnp.max(m.reshape(BNE, K, h), axis=1) + b2_ref[...])
    s_ref[...] = s

    @pl.when(i == 0)
    def _():
        st_ref[...] = jnp.zeros_like(st_ref)

    st_ref[0:1, :] += jnp.sum(s, axis=0, keepdims=True)
    st_ref[1:2, :] += jnp.sum(s * s, axis=0, keepdims=True)


def _edge(p, qg, b1, w2, b2, h, hpad, half):
    nb = NH // BNE
    return pl.pallas_call(
        _edge_body,
        grid=(nb,),
        in_specs=[
            pl.BlockSpec((BNE, hpad), lambda i: (i + half * nb, 0)),
            pl.BlockSpec((EBLK, hpad), lambda i: (i, 0)),
            pl.BlockSpec((1, hpad), lambda i: (0, 0)),
            pl.BlockSpec((hpad, h), lambda i: (0, 0)),
            pl.BlockSpec((1, h), lambda i: (0, 0)),
        ],
        out_specs=[
            pl.BlockSpec((BNE, h), lambda i: (i, 0)),
            pl.BlockSpec((8, h), lambda i: (0, 0)),
        ],
        out_shape=[
            jax.ShapeDtypeStruct((NH, h), jnp.float32),
            jax.ShapeDtypeStruct((8, h), jnp.float32),
        ],
    )(p, qg, b1, w2, b2)


def _heada_body(s1_ref, s2_ref, s3_ref, st1a_ref, st1b_ref, st2a_ref,
                st2b_ref, st3a_ref, st3b_ref,
                g1_ref, be1_ref, g2_ref, be2_ref, g3_ref, be3_ref,
                l1w_ref, l1b_ref, l2w_ref, l2b_ref, bf_ref,
                hl_ref, gm_ref):
    i = pl.program_id(0)

    def norm(s_ref, sta_ref, stb_ref, g_ref, be_ref):
        st = sta_ref[...] + stb_ref[...]
        mu = st[0:1, :] / N
        var = st[1:2, :] / N - mu * mu
        inv = lax.rsqrt(var + 1e-5)
        return (s_ref[...] - mu) * inv * g_ref[...] + be_ref[...]

    h1 = norm(s1_ref, st1a_ref, st1b_ref, g1_ref, be1_ref)
    h2 = norm(s2_ref, st2a_ref, st2b_ref, g2_ref, be2_ref)
    h3 = norm(s3_ref, st3a_ref, st3b_ref, g3_ref, be3_ref)
    acc = (jnp.dot(h1, l1w_ref[0:96, :], preferred_element_type=jnp.float32)
           + jnp.dot(h2, l1w_ref[96:256, :], preferred_element_type=jnp.float32)
           + jnp.dot(h3, l1w_ref[256:512, :], preferred_element_type=jnp.float32)
           + l1b_ref[...])
    hl = jax.nn.relu(acc)
    hl2 = jax.nn.relu(jnp.dot(hl, l2w_ref[...],
                              preferred_element_type=jnp.float32) + l2b_ref[...])
    hl_ref[...] = hl2

    @pl.when(i == 0)
    def _():
        rows = lax.broadcasted_iota(jnp.int32, (8, 256), 0)
        gm_ref[...] = jnp.where(rows < B, -1e30, 0.0)

    bf = bf_ref[...]
    for b in range(B):
        cand = jnp.max(jnp.where(bf == float(b), hl2, -1e30),
                       axis=0, keepdims=True)
        gm_ref[b:b + 1, :] = jnp.maximum(gm_ref[b:b + 1, :], cand)


def _heada(s1, s2, s3, sts, g1, be1, g2, be2, g3, be3,
           l1w, l1b, l2w, l2b, batch_f):
    bf = batch_f[:, None]
    full = lambda a: pl.BlockSpec(a.shape, lambda i: (0, 0))
    return pl.pallas_call(
        _heada_body,
        grid=(N // BN_,),
        in_specs=[
            pl.BlockSpec((BN_, 96), lambda i: (i, 0)),
            pl.BlockSpec((BN_, 160), lambda i: (i, 0)),
            pl.BlockSpec((BN_, 256), lambda i: (i, 0)),
        ] + [full(st) for st in sts] + [
            full(g1), full(be1), full(g2), full(be2), full(g3), full(be3),
            full(l1w), full(l1b), full(l2w), full(l2b),
            pl.BlockSpec((BN_, 1), lambda i: (i, 0)),
        ],
        out_specs=[
            pl.BlockSpec((BN_, 256), lambda i: (i, 0)),
            pl.BlockSpec((8, 256), lambda i: (0, 0)),
        ],
        out_shape=[
            jax.ShapeDtypeStruct((N, 256), jnp.float32),
            jax.ShapeDtypeStruct((8, 256), jnp.float32),
        ],
    )(s1, s2, s3, *sts, g1, be1, g2, be2, g3, be3,
      l1w, l1b, l2w, l2b, bf)


def _headb_body(gm_ref, pg_ref, gw_ref, gb_ref, pw_ref, pb_ref,
                lng_ref, lnb_ref, h1w_ref, h1b_ref, cg_ref):
    gg = jax.nn.relu(jnp.dot(gm_ref[...], gw_ref[...],
                             preferred_element_type=jnp.float32) + gb_ref[...])
    z = jnp.dot(pg_ref[...], pw_ref[...],
                preferred_element_type=jnp.float32) + pb_ref[...]
    mu = jnp.mean(z, axis=-1, keepdims=True)
    var = jnp.mean(z * z, axis=-1, keepdims=True) - mu * mu
    zn = (z - mu) * lax.rsqrt(var + 1e-5) * lng_ref[...] + lnb_ref[...]
    gs = zn * (1.0 / (1.0 + jnp.exp(-zn)))
    cg_ref[...] = (jnp.dot(gg, h1w_ref[256:512, :],
                           preferred_element_type=jnp.float32)
                   + jnp.dot(gs, h1w_ref[512:768, :],
                             preferred_element_type=jnp.float32)
                   + h1b_ref[...])


def _headb(gm, pg8, gw, gb, pw, pb, lng, lnb, h1w, h1b):
    full = lambda a: pl.BlockSpec(a.shape, lambda: (0, 0))
    return pl.pallas_call(
        _headb_body,
        in_specs=[full(gm), full(pg8), full(gw), full(gb), full(pw), full(pb),
                  full(lng), full(lnb), full(h1w), full(h1b)],
        out_specs=full(jnp.zeros((8, 256))),
        out_shape=jax.ShapeDtypeStruct((8, 256), jnp.float32),
    )(gm, pg8, gw, gb, pw, pb, lng, lnb, h1w, h1b)


def _headd_body(hl_ref, cg_ref, h1w_ref, h2w_ref, h2b_ref, bf_ref, out_ref):
    bf = bf_ref[...]
    sel = jnp.zeros((BN_, 256), jnp.float32)
    for b in range(B):
        sel = sel + jnp.where(bf == float(b), cg_ref[b:b + 1, :], 0.0)
    h4 = jax.nn.relu(jnp.dot(hl_ref[...], h1w_ref[0:256, :],
                             preferred_element_type=jnp.float32) + sel)
    out_ref[...] = jnp.dot(h4, h2w_ref[...],
                           preferred_element_type=jnp.float32) + h2b_ref[...]


def _headd(hl, cg, h1w, h2wp, h2bp, batch_f):
    bf = batch_f[:, None]
    full = lambda a: pl.BlockSpec(a.shape, lambda i: (0, 0))
    return pl.pallas_call(
        _headd_body,
        grid=(N // BN_,),
        in_specs=[
            pl.BlockSpec((BN_, 256), lambda i: (i, 0)),
            full(cg), full(h1w), full(h2wp), full(h2bp),
            pl.BlockSpec((BN_, 1), lambda i: (i, 0)),
        ],
        out_specs=pl.BlockSpec((BN_, 8), lambda i: (i, 0)),
        out_shape=jax.ShapeDtypeStruct((N, 8), jnp.float32),
    )(hl, cg, h1w, h2wp, h2bp, bf)


def kernel(xyz, x, pos_in, batch, pretrain_global, params):
    p = params
    batch_f = batch.astype(jnp.float32)

    col_a = _knn(pos_in, batch, batch_f, 0)
    col_b = _knn(pos_in, batch, batch_f, 1)
    tail = (jnp.arange(EHPAD - EH, dtype=jnp.int32) * 16) % N
    idx_a = jnp.concatenate([col_a.reshape(EH), tail])
    idx_b = jnp.concatenate([col_b.reshape(EH), tail])

    def padc(w, hp):
        return jnp.pad(w, ((0, 0), (0, hp - w.shape[1])))

    def padr(w, hp):
        return jnp.pad(w, ((0, hp - w.shape[0]), (0, 0)))

    def layer(pq, b1, w2, b2, h, hpad):
        pl_, ql_ = pq
        qga = _sc_gather(ql_, idx_a, hpad)
        qgb = _sc_gather(ql_, idx_b, hpad)
        b1p = padc(b1, hpad)
        w2p = padr(w2, hpad)
        sa, sta = _edge(pl_, qga, b1p, w2p, b2, h, hpad, 0)
        sb, stb = _edge(pl_, qgb, b1p, w2p, b2, h, hpad, 1)
        return jnp.concatenate([sa, sb], axis=0), sta, stb

    xin = jnp.pad(jnp.concatenate([xyz, x], axis=1), ((0, 0), (0, 6)))
    w0p = jnp.pad(p['W0'], ((0, 6), (0, 0)))
    pq1 = _pq1(xin, w0p, p['b0'][None, :], padc(p['e1w1'], 128), 128)
    s1, st1a, st1b = layer(pq1, p['e1b1'][None, :], p['e1w2'],
                           p['e1b2'][None, :], 96, 128)

    pq2 = _pq23(s1, st1a, st1b, p['e1g'][None, :], p['e1be'][None, :],
                padc(p['e2w1'], 256), 256)
    s2, st2a, st2b = layer(pq2, p['e2b1'][None, :], p['e2w2'],
                           p['e2b2'][None, :], 160, 256)

    pq3 = _pq23(s2, st2a, st2b, p['e2g'][None, :], p['e2be'][None, :],
                p['e3w1'], 256)
    s3, st3a, st3b = layer(pq3, p['e3b1'][None, :], p['e3w2'],
                           p['e3b2'][None, :], 256, 256)

    hl, gm = _heada(s1, s2, s3, [st1a, st1b, st2a, st2b, st3a, st3b],
                    p['e1g'][None, :], p['e1be'][None, :],
                    p['e2g'][None, :], p['e2be'][None, :],
                    p['e3g'][None, :], p['e3be'][None, :],
                    p['l1w'], p['l1b'][None, :], p['l2w'], p['l2b'][None, :],
                    batch_f)
    pg8 = jnp.pad(pretrain_global, ((0, 4), (0, 0)))
    cg = _headb(gm, pg8, p['gw'], p['gb'][None, :], p['pw'], p['pb'][None, :],
                p['lng'][None, :], p['lnb'][None, :], p['h1w'], p['h1b'][None, :])
    h2wp = jnp.pad(p['h2w'], ((0, 0), (0, 7)))
    h2bp = jnp.pad(p['h2b'][None, :], ((0, 0), (0, 7)))
    out = _headd(hl, cg, p['h1w'], h2wp, h2bp, batch_f)
    return out[:, 0]
